# baseline (speedup 1.0000x reference)
import jax
import jax.numpy as jnp
from jax.experimental import pallas as pl
from jax.experimental.pallas import tpu as pltpu

_NUM_EMB = 1024
_NUM_HEADS = 4
_DH = 256
_D = _NUM_HEADS * _DH
_KP = _DH + 128
_COMMITMENT_COST = 0.25
_BN = 256
_LOG2E = 1.4426950408889634


def _vq_kernel(x_ref, emb_ref, q_ref, codes_ref, loss_ref, ee_ref, e1_ref,
               e2_ref):
    i = pl.program_id(0)

    @pl.when(i == 0)
    def _init():
        for h in range(_NUM_HEADS):
            eh = emb_ref[h]
            ee_ref[h:h + 1, :] = jnp.sum(eh * eh, axis=1)[None, :]
            e1_ref[h] = eh + eh
            e2_ref[h, :, :_DH] = eh.astype(jnp.bfloat16)
            e2_ref[h, :, _DH:] = jnp.ones((_NUM_EMB, 128), jnp.bfloat16)
        loss_ref[...] = jnp.zeros_like(loss_ref)

    x = x_ref[...]
    revlanes = jax.lax.broadcasted_iota(jnp.int32, (_BN, _NUM_EMB), 1) ^ (
        _NUM_EMB - 1)
    loss_part = jnp.float32(0.0)
    code_cols = []
    for h in range(_NUM_HEADS):
        xh = x[:, h * _DH:(h + 1) * _DH]
        ip2 = jax.lax.dot_general(xh, e1_ref[h], (((1,), (1,)), ((), ())),
                                  preferred_element_type=jnp.float32)
        logits = ip2 - ee_ref[h:h + 1, :]
        m = jnp.max(logits, axis=1, keepdims=True)
        t = logits - m
        ti = jax.lax.bitcast_convert_type(t, jnp.int32) | revlanes
        code_cols.append((_NUM_EMB - 1) - jnp.max(ti, axis=1, keepdims=True))
        p = jnp.exp(t).astype(jnp.bfloat16)
        q2 = jax.lax.dot_general(p, e2_ref[h], (((1,), (0,)), ((), ())),
                                 preferred_element_type=jnp.float32)
        qh = q2[:, :_DH] * (1.0 / q2[:, _DH:_DH + 1])
        q_ref[:, h * _DH:(h + 1) * _DH] = qh
        dh = qh - xh
        loss_part += jnp.sum(dh * dh)
    codes_ref[...] = jnp.concatenate(code_cols, axis=1)

    loss_ref[...] += jnp.full(loss_ref.shape, loss_part, jnp.float32)


def kernel(inputs, emb):
    n = inputs.shape[0]
    q, codes, loss_acc = pl.pallas_call(
        _vq_kernel,
        grid=(n // _BN,),
        in_specs=[
            pl.BlockSpec((_BN, None, _D), lambda i: (i, 0, 0)),
            pl.BlockSpec((_NUM_HEADS, _NUM_EMB, _DH), lambda i: (0, 0, 0)),
        ],
        out_specs=[
            pl.BlockSpec((_BN, None, _D), lambda i: (i, 0, 0)),
            pl.BlockSpec((_BN, _NUM_HEADS), lambda i: (i, 0)),
            pl.BlockSpec((1, 1, 128), lambda i: (0, 0, 0)),
        ],
        out_shape=[
            jax.ShapeDtypeStruct((n, 1, _D), jnp.float32),
            jax.ShapeDtypeStruct((n, _NUM_HEADS), jnp.int32),
            jax.ShapeDtypeStruct((1, 1, 128), jnp.float32),
        ],
        scratch_shapes=[
            pltpu.VMEM((_NUM_HEADS, _NUM_EMB), jnp.float32),
            pltpu.VMEM((_NUM_HEADS, _NUM_EMB, _DH), jnp.float32),
            pltpu.VMEM((_NUM_HEADS, _NUM_EMB, _KP), jnp.bfloat16),
        ],
    )(inputs, emb)
    loss = loss_acc[0, 0, 0] * (_COMMITMENT_COST / (n * _D))
    return loss, q, codes


# speedup vs baseline: 1.2193x; 1.2193x over previous
import jax
import jax.numpy as jnp
from jax.experimental import pallas as pl
from jax.experimental.pallas import tpu as pltpu

_NUM_EMB = 1024
_NUM_HEADS = 4
_DH = 256
_D = _NUM_HEADS * _DH
_KP = _DH + 128
_COMMITMENT_COST = 0.25
_BN = 512
_LOG2E = 1.4426950408889634


def _vq_kernel(x_ref, emb_ref, q_ref, codes_ref, loss_ref, ee_ref, e1_ref,
               e2_ref):
    i = pl.program_id(0)

    @pl.when(i == 0)
    def _init():
        for h in range(_NUM_HEADS):
            eh = emb_ref[h]
            ee_ref[h:h + 1, :] = jnp.sum(eh * eh, axis=1)[None, :]
            e1_ref[h] = eh + eh
            e2_ref[h, :, :_DH] = eh.astype(jnp.bfloat16)
            e2_ref[h, :, _DH:] = jnp.ones((_NUM_EMB, 128), jnp.bfloat16)
        loss_ref[...] = jnp.zeros_like(loss_ref)

    x = x_ref[...]
    revlanes = jax.lax.broadcasted_iota(jnp.int32, (_BN, _NUM_EMB), 1) ^ (
        _NUM_EMB - 1)
    loss_part = jnp.float32(0.0)
    code_cols = []
    for h in range(_NUM_HEADS):
        xh = x[:, h * _DH:(h + 1) * _DH]
        ip2 = jax.lax.dot_general(xh, e1_ref[h], (((1,), (1,)), ((), ())),
                                  preferred_element_type=jnp.float32)
        logits = ip2 - ee_ref[h:h + 1, :]
        m = jnp.max(logits, axis=1, keepdims=True)
        t = logits - m
        ti = jax.lax.bitcast_convert_type(t, jnp.int32) | revlanes
        code_cols.append((_NUM_EMB - 1) - jnp.max(ti, axis=1, keepdims=True))
        p = jnp.exp(t).astype(jnp.bfloat16)
        q2 = jax.lax.dot_general(p, e2_ref[h], (((1,), (0,)), ((), ())),
                                 preferred_element_type=jnp.float32)
        qh = q2[:, :_DH] * (1.0 / q2[:, _DH:_DH + 1])
        q_ref[:, h * _DH:(h + 1) * _DH] = qh
        dh = qh - xh
        loss_part += jnp.sum(dh * dh)
    codes_ref[...] = jnp.concatenate(code_cols, axis=1)

    loss_ref[...] += jnp.full(loss_ref.shape, loss_part, jnp.float32)


def kernel(inputs, emb):
    n = inputs.shape[0]
    q, codes, loss_acc = pl.pallas_call(
        _vq_kernel,
        grid=(n // _BN,),
        in_specs=[
            pl.BlockSpec((_BN, None, _D), lambda i: (i, 0, 0)),
            pl.BlockSpec((_NUM_HEADS, _NUM_EMB, _DH), lambda i: (0, 0, 0)),
        ],
        out_specs=[
            pl.BlockSpec((_BN, None, _D), lambda i: (i, 0, 0)),
            pl.BlockSpec((_BN, _NUM_HEADS), lambda i: (i, 0)),
            pl.BlockSpec((1, 1, 128), lambda i: (0, 0, 0)),
        ],
        out_shape=[
            jax.ShapeDtypeStruct((n, 1, _D), jnp.float32),
            jax.ShapeDtypeStruct((n, _NUM_HEADS), jnp.int32),
            jax.ShapeDtypeStruct((1, 1, 128), jnp.float32),
        ],
        scratch_shapes=[
            pltpu.VMEM((_NUM_HEADS, _NUM_EMB), jnp.float32),
            pltpu.VMEM((_NUM_HEADS, _NUM_EMB, _DH), jnp.float32),
            pltpu.VMEM((_NUM_HEADS, _NUM_EMB, _KP), jnp.bfloat16),
        ],
    )(inputs, emb)
    loss = loss_acc[0, 0, 0] * (_COMMITMENT_COST / (n * _D))
    return loss, q, codes
